# baseline (speedup 1.0000x reference)
import jax
import jax.numpy as jnp
from jax.experimental import pallas as pl


def _lrelu(v, s):
    return jnp.where(v >= 0, v, s * v)


def _brow(b, k):
    return jnp.zeros((8, k), jnp.float32).at[0].set(b)


def _ceil_to(v, m):
    return ((v + m - 1) // m) * m


def _mm_plain_body(x_ref, w_ref, o_ref):
    o_ref[...] = jnp.dot(x_ref[...], w_ref[...],
                         preferred_element_type=jnp.float32)


def _mm_actbias_body(x_ref, w_ref, b_ref, o_ref):
    xa = _lrelu(x_ref[...] + b_ref[0:1, :], 0.01)
    o_ref[...] = jnp.dot(xa, w_ref[...], preferred_element_type=jnp.float32)


def _mm(x, w, bn=256, bias=None):
    r0 = x.shape[0]
    rp = _ceil_to(r0, bn)
    xp = jnp.pad(x, ((0, rp - r0), (0, 0))) if rp != r0 else x
    k = xp.shape[1]
    m = w.shape[1]
    grid = (rp // bn,)
    if bias is None:
        out = pl.pallas_call(
            _mm_plain_body,
            grid=grid,
            in_specs=[pl.BlockSpec((bn, k), lambda i: (i, 0)),
                      pl.BlockSpec((k, m), lambda i: (0, 0))],
            out_specs=pl.BlockSpec((bn, m), lambda i: (i, 0)),
            out_shape=jax.ShapeDtypeStruct((rp, m), jnp.float32),
        )(xp, w)
    else:
        out = pl.pallas_call(
            _mm_actbias_body,
            grid=grid,
            in_specs=[pl.BlockSpec((bn, k), lambda i: (i, 0)),
                      pl.BlockSpec((k, m), lambda i: (0, 0)),
                      pl.BlockSpec((8, k), lambda i: (0, 0))],
            out_specs=pl.BlockSpec((bn, m), lambda i: (i, 0)),
            out_shape=jax.ShapeDtypeStruct((rp, m), jnp.float32),
        )(xp, w, _brow(bias, k))
    return out[:r0]


def _ew3_body(a_ref, b_ref, c_ref, o_ref):
    o_ref[...] = _lrelu(a_ref[...] + b_ref[...] + c_ref[...], 0.2)


def _exp_body(a_ref, m_ref, o_ref):
    o_ref[...] = jnp.exp(a_ref[...] - m_ref[...])


def _ew_sheets(body, args, bw=128, br=8):
    r = args[0].shape[0]
    grid = (r // br,)
    spec = pl.BlockSpec((br, bw), lambda i: (i, 0))
    return pl.pallas_call(
        body, grid=grid,
        in_specs=[spec] * len(args),
        out_specs=spec,
        out_shape=jax.ShapeDtypeStruct((r, bw), jnp.float32),
    )(*args)


def _scale_body(h_ref, e_ref, o_ref):
    o_ref[...] = h_ref[...] * e_ref[:, 0:1]


def _divcol_body(h_ref, d_ref, o_ref):
    o_ref[...] = h_ref[...] / (d_ref[:, 0:1] + 1e-16)


def _rowscale(body, rows, colv, bn=512):
    r, m = rows.shape
    cv = jnp.broadcast_to(colv[:, None], (r, 8))
    grid = (r // bn,)
    return pl.pallas_call(
        body, grid=grid,
        in_specs=[pl.BlockSpec((bn, m), lambda i: (i, 0)),
                  pl.BlockSpec((bn, 8), lambda i: (i, 0))],
        out_specs=pl.BlockSpec((bn, m), lambda i: (i, 0)),
        out_shape=jax.ShapeDtypeStruct((r, m), jnp.float32),
    )(rows, cv)


def _edge_mlp_body(g1_ref, g2_ref, e3_ref, b1_ref, w2_ref, b2_ref,
                   tw1_ref, tb1_ref, vw1_ref, vb1_ref, wlast_ref, o_ref):
    c = _lrelu(g1_ref[...] + g2_ref[...] + e3_ref[...] + b1_ref[0:1, :], 0.01)
    c2 = _lrelu(jnp.dot(c, w2_ref[...], preferred_element_type=jnp.float32)
                + b2_ref[0:1, :], 0.01)
    t1 = _lrelu(jnp.dot(c2, tw1_ref[...], preferred_element_type=jnp.float32)
                + tb1_ref[0:1, :], 0.01)
    v1 = _lrelu(jnp.dot(c2, vw1_ref[...], preferred_element_type=jnp.float32)
                + vb1_ref[0:1, :], 0.01)
    tv2 = jnp.dot(jnp.concatenate([t1, v1], axis=1), wlast_ref[...],
                  preferred_element_type=jnp.float32)
    te = jax.nn.sigmoid(tv2[:, 0:1] + wlast_ref[0, 2])
    tv = tv2[:, 1:2] + wlast_ref[0, 3]
    o_ref[...] = jnp.concatenate(
        [te * tv, te, jnp.zeros_like(tv2[:, 2:8])], axis=1)


def _edge_mlp(g1s, g2s, e3, p, be=512):
    ep, hc4 = g1s.shape
    hc2, hc = hc4 // 2, hc4 // 4
    w2 = p['ef_w2'].T
    tw1 = p['tc_w1'].T
    vw1 = p['vr_w1'].T
    wlast = jnp.zeros((hc2, 8), jnp.float32)
    wlast = wlast.at[:hc, 0].set(p['tc_w2'][0])
    wlast = wlast.at[hc:, 1].set(p['vr_w2'][0])
    wlast = wlast.at[0, 2].set(p['tc_b2'][0])
    wlast = wlast.at[0, 3].set(p['vr_b2'][0])
    grid = (ep // be,)

    def bs(r, c):
        return pl.BlockSpec((r, c), lambda i: (i, 0))

    def const(r, c):
        return pl.BlockSpec((r, c), lambda i: (0, 0))

    return pl.pallas_call(
        _edge_mlp_body, grid=grid,
        in_specs=[bs(be, hc4), bs(be, hc4), bs(be, hc4),
                  const(8, hc4), const(hc4, hc2), const(8, hc2),
                  const(hc2, hc), const(8, hc),
                  const(hc2, hc), const(8, hc), const(hc2, 8)],
        out_specs=bs(be, 8),
        out_shape=jax.ShapeDtypeStruct((ep, 8), jnp.float32),
    )(g1s, g2s, e3, _brow(p['ef_b1'], hc4), w2,
      _brow(p['ef_b2'], hc2), tw1, _brow(p['tc_b1'], hc),
      vw1, _brow(p['vr_b1'], hc), wlast)


def _gat_layer(x, src2p, dst2p, dst2, ets, n, e2, prev_b, W, a_src, a_dst):
    o = W.shape[0]
    e2p = src2p.shape[0]
    ws = W.T @ a_src
    wd = W.T @ a_dst
    wt = jnp.concatenate(
        [W.T, ws[:, None], wd[:, None],
         jnp.zeros((W.shape[1], 6), jnp.float32)], axis=1)
    hsd = _mm(x, wt, bias=prev_b)
    hs_ext = hsd[src2p]
    d128 = jnp.pad(hsd[:, o + 1:o + 2], ((0, 0), (0, 127)))
    dg = d128[dst2p][:, 0]

    sg_sheet = hs_ext[:, o].reshape(-1, 128)
    dg_sheet = dg.reshape(-1, 128)
    al = _ew_sheets(_ew3_body, (sg_sheet, dg_sheet, ets))
    al_flat = al.reshape(-1)[:e2]
    amax = jax.ops.segment_max(al_flat, dst2, num_segments=n)
    amax = jnp.where(jnp.isfinite(amax), amax, 0.0)
    amax128 = jnp.pad(amax[:, None], ((0, 0), (0, 127)))
    amg = amax128[dst2p][:, 0]
    ex = _ew_sheets(_exp_body, (al, amg.reshape(-1, 128)))
    ex_flat = ex.reshape(-1)
    den = jax.ops.segment_sum(ex_flat[:e2], dst2, num_segments=n)

    hw = _rowscale(_scale_body, hs_ext, ex_flat)
    num = jax.ops.segment_sum(hw[:e2, :o], dst2, num_segments=n)
    nump = jnp.pad(num, ((0, _ceil_to(n, 512) - n), (0, 0)))
    denp = jnp.pad(den, (0, _ceil_to(n, 512) - n))
    return _rowscale(_divcol_body, nump, denp)[:n]


def kernel(x, edge_index, edge_attr, params):
    p = params
    n = x.shape[0]
    e = edge_index.shape[1]
    src, dst = edge_index[0], edge_index[1]
    loop = jnp.arange(n, dtype=src.dtype)
    src2 = jnp.concatenate([src, loop])
    dst2 = jnp.concatenate([dst, loop])
    e2 = e + n
    e2p = _ceil_to(e2, 1024)
    src2p = jnp.pad(src2, (0, e2p - e2))
    dst2p = jnp.pad(dst2, (0, e2p - e2))
    ea_mean = edge_attr.mean(axis=0)
    ea2 = jnp.concatenate(
        [edge_attr, jnp.broadcast_to(ea_mean, (n, edge_attr.shape[1]))], axis=0)

    ets = []
    for i in (1, 2, 3):
        v = p['We%d' % i].T @ p['ae%d' % i]
        ets.append(jnp.pad(ea2 @ v, (0, e2p - e2)).reshape(-1, 128))

    agg1 = _gat_layer(x, src2p, dst2p, dst2, ets[0], n, e2, None,
                      p['W1'], p['as1'], p['ad1'])
    agg2 = _gat_layer(agg1, src2p, dst2p, dst2, ets[1], n, e2, p['b1'],
                      p['W2'], p['as2'], p['ad2'])
    agg3 = _gat_layer(agg2, src2p, dst2p, dst2, ets[2], n, e2, p['b2'],
                      p['W3'], p['as3'], p['ad3'])

    hc4 = p['ef_w1'].shape[0]
    a1t = p['ef_w1'][:, :hc4].T
    a2t = p['ef_w1'][:, hc4:2 * hc4].T
    a3t = p['ef_w1'][:, 2 * hc4:].T
    g = _mm(agg3, jnp.concatenate([a1t, a2t], axis=1), bias=p['b3'])
    g1 = g[:, :hc4]
    g2 = g[:, hc4:]
    ep = _ceil_to(e, 512)
    srcp = jnp.pad(src, (0, ep - e))
    dstp = jnp.pad(dst, (0, ep - e))
    g1s = g1[srcp]
    g2s = g2[dstp]
    e3 = _mm(jnp.pad(edge_attr, ((0, ep - e), (0, 0))), a3t, bn=512)
    out = _edge_mlp(g1s, g2s, e3, p)[:e]
    return (out[:, 0:1], out[:, 1:2])


# speedup vs baseline: 1.3473x; 1.3473x over previous
import jax
import jax.numpy as jnp
from jax.experimental import pallas as pl


def _lrelu(v, s):
    return jnp.where(v >= 0, v, s * v)


def _brow(b, k):
    return jnp.zeros((8, k), jnp.float32).at[0].set(b)


def _ceil_to(v, m):
    return ((v + m - 1) // m) * m


def _mm_plain_body(x_ref, w_ref, o_ref):
    o_ref[...] = jnp.dot(x_ref[...], w_ref[...],
                         preferred_element_type=jnp.float32)


def _mm_actbias_body(x_ref, w_ref, b_ref, o_ref):
    xa = _lrelu(x_ref[...] + b_ref[0:1, :], 0.01)
    o_ref[...] = jnp.dot(xa, w_ref[...], preferred_element_type=jnp.float32)


def _mm(x, w, bn=256, bias=None):
    r0 = x.shape[0]
    rp = _ceil_to(r0, bn)
    xp = jnp.pad(x, ((0, rp - r0), (0, 0))) if rp != r0 else x
    k = xp.shape[1]
    m = w.shape[1]
    grid = (rp // bn,)
    if bias is None:
        out = pl.pallas_call(
            _mm_plain_body,
            grid=grid,
            in_specs=[pl.BlockSpec((bn, k), lambda i: (i, 0)),
                      pl.BlockSpec((k, m), lambda i: (0, 0))],
            out_specs=pl.BlockSpec((bn, m), lambda i: (i, 0)),
            out_shape=jax.ShapeDtypeStruct((rp, m), jnp.float32),
        )(xp, w)
    else:
        out = pl.pallas_call(
            _mm_actbias_body,
            grid=grid,
            in_specs=[pl.BlockSpec((bn, k), lambda i: (i, 0)),
                      pl.BlockSpec((k, m), lambda i: (0, 0)),
                      pl.BlockSpec((8, k), lambda i: (0, 0))],
            out_specs=pl.BlockSpec((bn, m), lambda i: (i, 0)),
            out_shape=jax.ShapeDtypeStruct((rp, m), jnp.float32),
        )(xp, w, _brow(bias, k))
    return out[:r0]


def _edge_mlp_body(g1_ref, g2_ref, e3_ref, b1_ref, w2_ref, b2_ref,
                   tw1_ref, tb1_ref, vw1_ref, vb1_ref, wlast_ref, o_ref):
    c = _lrelu(g1_ref[...] + g2_ref[...] + e3_ref[...] + b1_ref[0:1, :], 0.01)
    c2 = _lrelu(jnp.dot(c, w2_ref[...], preferred_element_type=jnp.float32)
                + b2_ref[0:1, :], 0.01)
    t1 = _lrelu(jnp.dot(c2, tw1_ref[...], preferred_element_type=jnp.float32)
                + tb1_ref[0:1, :], 0.01)
    v1 = _lrelu(jnp.dot(c2, vw1_ref[...], preferred_element_type=jnp.float32)
                + vb1_ref[0:1, :], 0.01)
    tv2 = jnp.dot(jnp.concatenate([t1, v1], axis=1), wlast_ref[...],
                  preferred_element_type=jnp.float32)
    te = jax.nn.sigmoid(tv2[:, 0:1] + wlast_ref[0, 2])
    tv = tv2[:, 1:2] + wlast_ref[0, 3]
    o_ref[...] = jnp.concatenate(
        [te * tv, te, jnp.zeros_like(tv2[:, 2:8])], axis=1)


def _edge_mlp(g1s, g2s, e3, p, be=512):
    ep, hc4 = g1s.shape
    hc2, hc = hc4 // 2, hc4 // 4
    w2 = p['ef_w2'].T
    tw1 = p['tc_w1'].T
    vw1 = p['vr_w1'].T
    wlast = jnp.zeros((hc2, 8), jnp.float32)
    wlast = wlast.at[:hc, 0].set(p['tc_w2'][0])
    wlast = wlast.at[hc:, 1].set(p['vr_w2'][0])
    wlast = wlast.at[0, 2].set(p['tc_b2'][0])
    wlast = wlast.at[0, 3].set(p['vr_b2'][0])
    grid = (ep // be,)

    def bs(r, c):
        return pl.BlockSpec((r, c), lambda i: (i, 0))

    def const(r, c):
        return pl.BlockSpec((r, c), lambda i: (0, 0))

    return pl.pallas_call(
        _edge_mlp_body, grid=grid,
        in_specs=[bs(be, hc4), bs(be, hc4), bs(be, hc4),
                  const(8, hc4), const(hc4, hc2), const(8, hc2),
                  const(hc2, hc), const(8, hc),
                  const(hc2, hc), const(8, hc), const(hc2, 8)],
        out_specs=bs(be, 8),
        out_shape=jax.ShapeDtypeStruct((ep, 8), jnp.float32),
    )(g1s, g2s, e3, _brow(p['ef_b1'], hc4), w2,
      _brow(p['ef_b2'], hc2), tw1, _brow(p['tc_b1'], hc),
      vw1, _brow(p['vr_b1'], hc), wlast)


def _gat_layer(x, src2p, dst2p, dst2, ets, n, e2, prev_b, W, a_src, a_dst):
    o = W.shape[0]
    e2p = src2p.shape[0]
    ws = W.T @ a_src
    wd = W.T @ a_dst
    wt = jnp.concatenate(
        [W.T, ws[:, None], wd[:, None],
         jnp.zeros((W.shape[1], 6), jnp.float32)], axis=1)
    hsd = _mm(x, wt, bias=prev_b)
    hs_ext = hsd[src2p]
    d128 = jnp.pad(hsd[:, o + 1:o + 2], ((0, 0), (0, 127)))
    dg = d128[dst2p][:, 0]

    al = _lrelu(hs_ext[:, o] + dg + ets, 0.2)
    al_flat = al[:e2]
    amax = jax.ops.segment_max(al_flat, dst2, num_segments=n)
    amax = jnp.where(jnp.isfinite(amax), amax, 0.0)
    amax128 = jnp.pad(amax[:, None], ((0, 0), (0, 127)))
    amg = amax128[dst2p][:, 0]
    ex = jnp.exp(al - amg)
    den = jax.ops.segment_sum(ex[:e2], dst2, num_segments=n)

    hw = hs_ext[:e2, :o] * ex[:e2, None]
    num = jax.ops.segment_sum(hw, dst2, num_segments=n)
    return num / (den[:, None] + 1e-16)


def kernel(x, edge_index, edge_attr, params):
    p = params
    n = x.shape[0]
    e = edge_index.shape[1]
    src, dst = edge_index[0], edge_index[1]
    loop = jnp.arange(n, dtype=src.dtype)
    src2 = jnp.concatenate([src, loop])
    dst2 = jnp.concatenate([dst, loop])
    e2 = e + n
    e2p = _ceil_to(e2, 1024)
    src2p = jnp.pad(src2, (0, e2p - e2))
    dst2p = jnp.pad(dst2, (0, e2p - e2))
    ea_mean = edge_attr.mean(axis=0)
    ea2 = jnp.concatenate(
        [edge_attr, jnp.broadcast_to(ea_mean, (n, edge_attr.shape[1]))], axis=0)

    ets = []
    for i in (1, 2, 3):
        v = p['We%d' % i].T @ p['ae%d' % i]
        ets.append(jnp.pad(ea2 @ v, (0, e2p - e2)))

    agg1 = _gat_layer(x, src2p, dst2p, dst2, ets[0], n, e2, None,
                      p['W1'], p['as1'], p['ad1'])
    agg2 = _gat_layer(agg1, src2p, dst2p, dst2, ets[1], n, e2, p['b1'],
                      p['W2'], p['as2'], p['ad2'])
    agg3 = _gat_layer(agg2, src2p, dst2p, dst2, ets[2], n, e2, p['b2'],
                      p['W3'], p['as3'], p['ad3'])

    hc4 = p['ef_w1'].shape[0]
    a1t = p['ef_w1'][:, :hc4].T
    a2t = p['ef_w1'][:, hc4:2 * hc4].T
    a3t = p['ef_w1'][:, 2 * hc4:].T
    g = _mm(agg3, jnp.concatenate([a1t, a2t], axis=1), bias=p['b3'])
    g1 = g[:, :hc4]
    g2 = g[:, hc4:]
    ep = _ceil_to(e, 512)
    srcp = jnp.pad(src, (0, ep - e))
    dstp = jnp.pad(dst, (0, ep - e))
    g1s = g1[srcp]
    g2s = g2[dstp]
    e3 = _mm(jnp.pad(edge_attr, ((0, ep - e), (0, 0))), a3t, bn=512)
    out = _edge_mlp(g1s, g2s, e3, p)[:e]
    return (out[:, 0:1], out[:, 1:2])


# speedup vs baseline: 1.3474x; 1.0000x over previous
import jax
import jax.numpy as jnp
from jax.experimental import pallas as pl


def _lrelu(v, s):
    return jnp.where(v >= 0, v, s * v)


def _brow(b, k):
    return jnp.zeros((8, k), jnp.float32).at[0].set(b)


def _ceil_to(v, m):
    return ((v + m - 1) // m) * m


def _mm_plain_body(x_ref, w_ref, o_ref):
    o_ref[...] = jnp.dot(x_ref[...], w_ref[...],
                         preferred_element_type=jnp.float32)


def _mm_actbias_body(x_ref, w_ref, b_ref, o_ref):
    xa = _lrelu(x_ref[...] + b_ref[0:1, :], 0.01)
    o_ref[...] = jnp.dot(xa, w_ref[...], preferred_element_type=jnp.float32)


def _mm(x, w, bn=256, bias=None):
    r0 = x.shape[0]
    rp = _ceil_to(r0, bn)
    xp = jnp.pad(x, ((0, rp - r0), (0, 0))) if rp != r0 else x
    k = xp.shape[1]
    m = w.shape[1]
    grid = (rp // bn,)
    if bias is None:
        out = pl.pallas_call(
            _mm_plain_body,
            grid=grid,
            in_specs=[pl.BlockSpec((bn, k), lambda i: (i, 0)),
                      pl.BlockSpec((k, m), lambda i: (0, 0))],
            out_specs=pl.BlockSpec((bn, m), lambda i: (i, 0)),
            out_shape=jax.ShapeDtypeStruct((rp, m), jnp.float32),
        )(xp, w)
    else:
        out = pl.pallas_call(
            _mm_actbias_body,
            grid=grid,
            in_specs=[pl.BlockSpec((bn, k), lambda i: (i, 0)),
                      pl.BlockSpec((k, m), lambda i: (0, 0)),
                      pl.BlockSpec((8, k), lambda i: (0, 0))],
            out_specs=pl.BlockSpec((bn, m), lambda i: (i, 0)),
            out_shape=jax.ShapeDtypeStruct((rp, m), jnp.float32),
        )(xp, w, _brow(bias, k))
    return out[:r0]


def _edge_mlp_body(g1_ref, g2_ref, e3_ref, b1_ref, w2_ref, b2_ref,
                   tw1_ref, tb1_ref, vw1_ref, vb1_ref, wlast_ref, o_ref):
    c = _lrelu(g1_ref[...] + g2_ref[...] + e3_ref[...] + b1_ref[0:1, :], 0.01)
    c2 = _lrelu(jnp.dot(c, w2_ref[...], preferred_element_type=jnp.float32)
                + b2_ref[0:1, :], 0.01)
    t1 = _lrelu(jnp.dot(c2, tw1_ref[...], preferred_element_type=jnp.float32)
                + tb1_ref[0:1, :], 0.01)
    v1 = _lrelu(jnp.dot(c2, vw1_ref[...], preferred_element_type=jnp.float32)
                + vb1_ref[0:1, :], 0.01)
    tv2 = jnp.dot(jnp.concatenate([t1, v1], axis=1), wlast_ref[...],
                  preferred_element_type=jnp.float32)
    te = jax.nn.sigmoid(tv2[:, 0:1] + wlast_ref[0, 2])
    tv = tv2[:, 1:2] + wlast_ref[0, 3]
    o_ref[...] = jnp.concatenate(
        [te * tv, te, jnp.zeros_like(tv2[:, 2:8])], axis=1)


def _edge_mlp(g1s, g2s, e3, p, be=512):
    ep, hc4 = g1s.shape
    hc2, hc = hc4 // 2, hc4 // 4
    w2 = p['ef_w2'].T
    tw1 = p['tc_w1'].T
    vw1 = p['vr_w1'].T
    wlast = jnp.zeros((hc2, 8), jnp.float32)
    wlast = wlast.at[:hc, 0].set(p['tc_w2'][0])
    wlast = wlast.at[hc:, 1].set(p['vr_w2'][0])
    wlast = wlast.at[0, 2].set(p['tc_b2'][0])
    wlast = wlast.at[0, 3].set(p['vr_b2'][0])
    grid = (ep // be,)

    def bs(r, c):
        return pl.BlockSpec((r, c), lambda i: (i, 0))

    def const(r, c):
        return pl.BlockSpec((r, c), lambda i: (0, 0))

    return pl.pallas_call(
        _edge_mlp_body, grid=grid,
        in_specs=[bs(be, hc4), bs(be, hc4), bs(be, hc4),
                  const(8, hc4), const(hc4, hc2), const(8, hc2),
                  const(hc2, hc), const(8, hc),
                  const(hc2, hc), const(8, hc), const(hc2, 8)],
        out_specs=bs(be, 8),
        out_shape=jax.ShapeDtypeStruct((ep, 8), jnp.float32),
    )(g1s, g2s, e3, _brow(p['ef_b1'], hc4), w2,
      _brow(p['ef_b2'], hc2), tw1, _brow(p['tc_b1'], hc),
      vw1, _brow(p['vr_b1'], hc), wlast)


def _gat_layer(x, src2p, dst2p, dst2, ets, n, e2, prev_b, W, a_src, a_dst):
    o = W.shape[0]
    e2p = src2p.shape[0]
    ws = W.T @ a_src
    wd = W.T @ a_dst
    wt = jnp.concatenate(
        [W.T, ws[:, None], wd[:, None],
         jnp.zeros((W.shape[1], 6), jnp.float32)], axis=1)
    hsd = _mm(x, wt, bias=prev_b)
    hs_ext = hsd[src2p]
    d128 = jnp.pad(hsd[:, o + 1:o + 2], ((0, 0), (0, 7)))
    dg = d128[dst2p][:, 0]

    al = _lrelu(hs_ext[:, o] + dg + ets, 0.2)
    al_flat = al[:e2]
    amax = jax.ops.segment_max(al_flat, dst2, num_segments=n)
    amax = jnp.where(jnp.isfinite(amax), amax, 0.0)
    amax128 = jnp.pad(amax[:, None], ((0, 0), (0, 7)))
    amg = amax128[dst2p][:, 0]
    ex = jnp.exp(al - amg)
    den = jax.ops.segment_sum(ex[:e2], dst2, num_segments=n)

    hw = hs_ext[:e2, :o] * ex[:e2, None]
    num = jax.ops.segment_sum(hw, dst2, num_segments=n)
    return num / (den[:, None] + 1e-16)


def kernel(x, edge_index, edge_attr, params):
    p = params
    n = x.shape[0]
    e = edge_index.shape[1]
    src, dst = edge_index[0], edge_index[1]
    loop = jnp.arange(n, dtype=src.dtype)
    src2 = jnp.concatenate([src, loop])
    dst2 = jnp.concatenate([dst, loop])
    e2 = e + n
    e2p = _ceil_to(e2, 1024)
    src2p = jnp.pad(src2, (0, e2p - e2))
    dst2p = jnp.pad(dst2, (0, e2p - e2))
    ea_mean = edge_attr.mean(axis=0)
    ea2 = jnp.concatenate(
        [edge_attr, jnp.broadcast_to(ea_mean, (n, edge_attr.shape[1]))], axis=0)

    ets = []
    for i in (1, 2, 3):
        v = p['We%d' % i].T @ p['ae%d' % i]
        ets.append(jnp.pad(ea2 @ v, (0, e2p - e2)))

    agg1 = _gat_layer(x, src2p, dst2p, dst2, ets[0], n, e2, None,
                      p['W1'], p['as1'], p['ad1'])
    agg2 = _gat_layer(agg1, src2p, dst2p, dst2, ets[1], n, e2, p['b1'],
                      p['W2'], p['as2'], p['ad2'])
    agg3 = _gat_layer(agg2, src2p, dst2p, dst2, ets[2], n, e2, p['b2'],
                      p['W3'], p['as3'], p['ad3'])

    hc4 = p['ef_w1'].shape[0]
    a1t = p['ef_w1'][:, :hc4].T
    a2t = p['ef_w1'][:, hc4:2 * hc4].T
    a3t = p['ef_w1'][:, 2 * hc4:].T
    g = _mm(agg3, jnp.concatenate([a1t, a2t], axis=1), bias=p['b3'])
    g1 = g[:, :hc4]
    g2 = g[:, hc4:]
    ep = _ceil_to(e, 512)
    srcp = jnp.pad(src, (0, ep - e))
    dstp = jnp.pad(dst, (0, ep - e))
    g1s = g1[srcp]
    g2s = g2[dstp]
    e3 = _mm(jnp.pad(edge_attr, ((0, ep - e), (0, 0))), a3t, bn=512)
    out = _edge_mlp(g1s, g2s, e3, p)[:e]
    return (out[:, 0:1], out[:, 1:2])
